# baseline (speedup 1.0000x reference)
import jax
import jax.numpy as jnp
from jax.experimental import pallas as pl
from jax.experimental.pallas import tpu as pltpu

B, T, N, H = 4, 16, 10000, 64
R = B * T
W = N * H
C = 4
WC = W // C


def _body(sp_ref, stack_ref, hid_ref, hint_ref, spv_ref, out_ref, ptr_ref):
    c = pl.program_id(0)
    r = pl.program_id(1)
    b = r // T
    tgt = T * b + sp_ref[b] + 1

    @pl.when(r == tgt)
    def _():
        out_ref[...] = hid_ref[...]

    @pl.when(r != tgt)
    def _():
        out_ref[...] = stack_ref[...]

    @pl.when((c == 0) & (r == 0))
    def _():
        h = hint_ref[...]
        a0 = h[:, :, 0]
        a1 = h[:, :, 1]
        a2 = h[:, :, 2]
        ops = jnp.where(a0 >= a1,
                        jnp.where(a0 >= a2, 0, 2),
                        jnp.where(a1 >= a2, 1, 2)).astype(jnp.int32)
        ptr_ref[...] = jnp.maximum(spv_ref[...] + ops - 1, 0)


def kernel(stack, stack_pointers, hint_preds, hiddens, graph_fts):
    del graph_fts
    sp_flat = jnp.reshape(stack_pointers, (B,))
    stack2 = jnp.reshape(stack, (R, 1, W))
    hid2 = jnp.reshape(hiddens, (B, 1, W))

    def stack_idx(c, r, sp):
        b = r // T
        rr = jnp.where(r == T * b + sp[b] + 1, r - 1, r)
        return (rr, 0, c)

    grid_spec = pltpu.PrefetchScalarGridSpec(
        num_scalar_prefetch=1,
        grid=(C, R),
        in_specs=[
            pl.BlockSpec((1, 1, WC), stack_idx),
            pl.BlockSpec((1, 1, WC), lambda c, r, sp: (r // T, 0, c)),
            pl.BlockSpec((1, B, 3), lambda c, r, sp: (0, 0, 0)),
            pl.BlockSpec((1, B), lambda c, r, sp: (0, 0)),
        ],
        out_specs=[
            pl.BlockSpec((1, 1, WC), lambda c, r, sp: (r, 0, c)),
            pl.BlockSpec((1, B), lambda c, r, sp: (0, 0)),
        ],
    )

    new_stack2, new_ptrs = pl.pallas_call(
        _body,
        grid_spec=grid_spec,
        out_shape=[
            jax.ShapeDtypeStruct((R, 1, W), jnp.float32),
            jax.ShapeDtypeStruct((1, B), jnp.int32),
        ],
    )(sp_flat, stack2, hid2, hint_preds, stack_pointers)
    return (jnp.reshape(new_stack2, (B, T, N, H)), new_ptrs)


# speedup vs baseline: 2.1771x; 2.1771x over previous
import jax
import jax.numpy as jnp
from jax.experimental import pallas as pl
from jax.experimental.pallas import tpu as pltpu

B, T, N, H = 4, 16, 10000, 64
S = 5
NS = N // S


def _body(sp_ref, stack_ref, hid_ref, hint_ref, spv_ref, out_ref, ptr_ref):
    b = pl.program_id(0)
    s = pl.program_id(1)
    t = pl.program_id(2)
    tgt = sp_ref[b] + 1

    @pl.when(t == tgt)
    def _():
        out_ref[...] = hid_ref[...]

    @pl.when(t != tgt)
    def _():
        out_ref[...] = stack_ref[...]

    @pl.when((b == 0) & (s == 0) & (t == 0))
    def _():
        h = hint_ref[...]
        a0 = h[:, :, 0]
        a1 = h[:, :, 1]
        a2 = h[:, :, 2]
        ops = jnp.where(a0 >= a1,
                        jnp.where(a0 >= a2, 0, 2),
                        jnp.where(a1 >= a2, 1, 2)).astype(jnp.int32)
        ptr_ref[...] = jnp.maximum(spv_ref[...] + ops - 1, 0)


def kernel(stack, stack_pointers, hint_preds, hiddens, graph_fts):
    del graph_fts
    sp_flat = jnp.reshape(stack_pointers, (B,))

    def stack_idx(b, s, t, sp):
        tt = jnp.where(t == sp[b] + 1, t - 1, t)
        return (b, tt, s, 0)

    grid_spec = pltpu.PrefetchScalarGridSpec(
        num_scalar_prefetch=1,
        grid=(B, S, T),
        in_specs=[
            pl.BlockSpec((1, 1, NS, H), stack_idx),
            pl.BlockSpec((1, 1, NS, H), lambda b, s, t, sp: (0, b, s, 0)),
            pl.BlockSpec((1, B, 3), lambda b, s, t, sp: (0, 0, 0)),
            pl.BlockSpec((1, B), lambda b, s, t, sp: (0, 0)),
        ],
        out_specs=[
            pl.BlockSpec((1, 1, NS, H), lambda b, s, t, sp: (b, t, s, 0)),
            pl.BlockSpec((1, B), lambda b, s, t, sp: (0, 0)),
        ],
    )

    new_stack, new_ptrs = pl.pallas_call(
        _body,
        grid_spec=grid_spec,
        out_shape=[
            jax.ShapeDtypeStruct((B, T, N, H), jnp.float32),
            jax.ShapeDtypeStruct((1, B), jnp.int32),
        ],
    )(sp_flat, stack, hiddens, hint_preds, stack_pointers)
    return (new_stack, new_ptrs)


# speedup vs baseline: 2.5578x; 1.1749x over previous
import jax
import jax.numpy as jnp
from jax.experimental import pallas as pl
from jax.experimental.pallas import tpu as pltpu

B, T, N, H = 4, 16, 10000, 64
S = 1
NS = N // S


def _body(sp_ref, stack_ref, hid_ref, hint_ref, spv_ref, out_ref, ptr_ref):
    b = pl.program_id(0)
    s = pl.program_id(1)
    t = pl.program_id(2)
    tgt = sp_ref[b] + 1

    @pl.when(t == tgt)
    def _():
        out_ref[...] = hid_ref[...]

    @pl.when(t != tgt)
    def _():
        out_ref[...] = stack_ref[...]

    @pl.when((b == 0) & (s == 0) & (t == 0))
    def _():
        h = hint_ref[...]
        a0 = h[:, :, 0]
        a1 = h[:, :, 1]
        a2 = h[:, :, 2]
        ops = jnp.where(a0 >= a1,
                        jnp.where(a0 >= a2, 0, 2),
                        jnp.where(a1 >= a2, 1, 2)).astype(jnp.int32)
        ptr_ref[...] = jnp.maximum(spv_ref[...] + ops - 1, 0)


def kernel(stack, stack_pointers, hint_preds, hiddens, graph_fts):
    del graph_fts
    sp_flat = jnp.reshape(stack_pointers, (B,))

    def stack_idx(b, s, t, sp):
        tt = jnp.where(t == sp[b] + 1, t - 1, t)
        return (b, tt, s, 0)

    grid_spec = pltpu.PrefetchScalarGridSpec(
        num_scalar_prefetch=1,
        grid=(B, S, T),
        in_specs=[
            pl.BlockSpec((1, 1, NS, H), stack_idx),
            pl.BlockSpec((1, 1, NS, H), lambda b, s, t, sp: (0, b, s, 0)),
            pl.BlockSpec((1, B, 3), lambda b, s, t, sp: (0, 0, 0)),
            pl.BlockSpec((1, B), lambda b, s, t, sp: (0, 0)),
        ],
        out_specs=[
            pl.BlockSpec((1, 1, NS, H), lambda b, s, t, sp: (b, t, s, 0)),
            pl.BlockSpec((1, B), lambda b, s, t, sp: (0, 0)),
        ],
    )

    new_stack, new_ptrs = pl.pallas_call(
        _body,
        grid_spec=grid_spec,
        out_shape=[
            jax.ShapeDtypeStruct((B, T, N, H), jnp.float32),
            jax.ShapeDtypeStruct((1, B), jnp.int32),
        ],
    )(sp_flat, stack, hiddens, hint_preds, stack_pointers)
    return (new_stack, new_ptrs)
